# baseline (speedup 1.0000x reference)
import functools

import jax
import jax.numpy as jnp
from jax import lax
from jax.experimental import pallas as pl
from jax.experimental.pallas import tpu as pltpu
from jax.experimental.pallas import tpu_sc as plsc

_NC = 2
_NS = 16
_NW = _NC * _NS

_GRP = 256


@functools.lru_cache(maxsize=None)
def _build_gather(n_rows: int, vocab: int, d: int, grp_per_super: int):
    assert n_rows % (_NW * _GRP) == 0
    grp_per_w = n_rows // (_NW * _GRP)
    assert grp_per_w % grp_per_super == 0
    n_super = grp_per_w // grp_per_super
    rows_per_super = grp_per_super * _GRP
    rows_per_w = grp_per_w * _GRP

    mesh = plsc.VectorSubcoreMesh(
        core_axis_name="c", subcore_axis_name="s",
        num_cores=_NC, num_subcores=_NS)

    assert n_super >= 2 and n_super % 2 == 0

    @functools.partial(
        pl.kernel,
        out_type=jax.ShapeDtypeStruct((n_rows, d), jnp.float32),
        mesh=mesh,
        scratch_types=[
            pltpu.VMEM((grp_per_w, _GRP), jnp.int32),
            pltpu.VMEM((2, rows_per_super, d), jnp.float32),
            pltpu.SemaphoreType.DMA((2,)),
            pltpu.SemaphoreType.DMA((2,)),
        ],
        compiler_params=pltpu.CompilerParams(use_tc_tiling_on_sc=False),
    )
    def k(idx_hbm, table_hbm, out_hbm, idx_v, rows_v, gsem, ssem):
        wid = lax.axis_index("s") * _NC + lax.axis_index("c")
        gbase = wid * grp_per_w
        rbase = wid * rows_per_w

        pltpu.sync_copy(idx_hbm.at[pl.ds(gbase, grp_per_w)], idx_v)

        def issue_gathers(s, b):
            for g in range(grp_per_super):
                pltpu.async_copy(
                    table_hbm.at[idx_v.at[s * grp_per_super + g]],
                    rows_v.at[b].at[pl.ds(g * _GRP, _GRP)],
                    gsem.at[b])

        def drain_gathers(b):
            pltpu.make_async_copy(
                out_hbm.at[pl.ds(0, rows_per_super)], rows_v.at[b],
                gsem.at[b]).wait()

        def issue_store(s, b):
            pltpu.async_copy(
                rows_v.at[b],
                out_hbm.at[pl.ds(rbase + s * rows_per_super, rows_per_super)],
                ssem.at[b])

        def drain_store(b):
            pltpu.make_async_copy(
                rows_v.at[b], out_hbm.at[pl.ds(0, rows_per_super)],
                ssem.at[b]).wait()

        issue_gathers(0, 0)
        issue_gathers(1, 1)

        @pl.loop(0, n_super - 2, step=2)
        def _steady(s):
            for b in range(2):
                drain_gathers(b)
                issue_store(s + b, b)
            for b in range(2):
                drain_store(b)
                issue_gathers(s + 2 + b, b)

        for b in range(2):
            drain_gathers(b)
            issue_store(n_super - 2 + b, b)
        for b in range(2):
            drain_store(b)

    return k


def kernel(input_seqs, table):
    batch, hist = input_seqs.shape
    vocab, d = table.shape
    n_rows = batch * hist
    idx2d = input_seqs.astype(jnp.int32).reshape(n_rows // _GRP, _GRP)
    out = _build_gather(n_rows, vocab, d, 5)(idx2d, table)
    return out.reshape(batch, hist, d)


# speedup vs baseline: 1.0008x; 1.0008x over previous
import functools

import jax
import jax.numpy as jnp
from jax import lax
from jax.experimental import pallas as pl
from jax.experimental.pallas import tpu as pltpu
from jax.experimental.pallas import tpu_sc as plsc

_NC = 2
_NS = 16
_NW = _NC * _NS


@functools.lru_cache(maxsize=None)
def _build_gather(b_sz: int, hist: int, vocab: int, d: int, nbuf: int):
    assert b_sz % _NW == 0
    rows_per_w = b_sz // _NW
    assert rows_per_w % nbuf == 0 and rows_per_w // nbuf >= 2

    mesh = plsc.VectorSubcoreMesh(
        core_axis_name="c", subcore_axis_name="s",
        num_cores=_NC, num_subcores=_NS)

    @functools.partial(
        pl.kernel,
        out_type=jax.ShapeDtypeStruct((b_sz, hist, d), jnp.float32),
        mesh=mesh,
        scratch_types=[
            pltpu.VMEM((rows_per_w, hist), jnp.int32),
            pltpu.VMEM((nbuf, hist, d), jnp.float32),
            pltpu.SemaphoreType.DMA((nbuf,)),
            pltpu.SemaphoreType.DMA((nbuf,)),
        ],
        compiler_params=pltpu.CompilerParams(use_tc_tiling_on_sc=False),
    )
    def k(idx_hbm, table_hbm, out_hbm, idx_v, rows_v, gsem, ssem):
        wid = lax.axis_index("s") * _NC + lax.axis_index("c")
        bbase = wid * rows_per_w

        pltpu.sync_copy(idx_hbm.at[pl.ds(bbase, rows_per_w)], idx_v)

        def gfire(i, b):
            pltpu.async_copy(
                table_hbm.at[idx_v.at[i]], rows_v.at[b], gsem.at[b])

        def gdrain(b):
            pltpu.make_async_copy(
                out_hbm.at[0], rows_v.at[b], gsem.at[b]).wait()

        def sfire(i, b):
            pltpu.async_copy(
                rows_v.at[b], out_hbm.at[bbase + i], ssem.at[b])

        def sdrain(b):
            pltpu.make_async_copy(
                rows_v.at[b], out_hbm.at[0], ssem.at[b]).wait()

        for b in range(nbuf):
            gfire(b, b)

        @pl.loop(0, rows_per_w - nbuf, step=nbuf)
        def _steady(i):
            for b in range(nbuf):
                gdrain(b)
                sfire(i + b, b)
            for b in range(nbuf):
                sdrain(b)
                gfire(i + nbuf + b, b)

        for b in range(nbuf):
            gdrain(b)
            sfire(rows_per_w - nbuf + b, b)
        for b in range(nbuf):
            sdrain(b)

    return k


def kernel(input_seqs, table):
    batch, hist = input_seqs.shape
    vocab, d = table.shape
    idx = input_seqs.astype(jnp.int32)
    return _build_gather(batch, hist, vocab, d, 4)(idx, table)
